# baseline (speedup 1.0000x reference)
import functools

import jax
import jax.numpy as jnp
from jax import lax
from jax.experimental import pallas as pl
from jax.experimental.pallas import tpu as pltpu
from jax.experimental.pallas import tpu_sc as plsc

_EPS = 1e-5

_NC = 2
_NS = 16
_NW = _NC * _NS

_K = 80
_ZR = 80
_NB = 4
_NSR = 8


def _sc_aggregate(feature, edges3):
    n, d = feature.shape
    _, nchunk, _ = edges3.shape
    nrch = n // _ZR
    nrch_per_tile = (nrch + _NS - 1) // _NS

    mesh = plsc.VectorSubcoreMesh(core_axis_name="c", subcore_axis_name="s")

    @functools.partial(
        pl.kernel,
        out_type=jax.ShapeDtypeStruct((_NC * n, d), jnp.float32),
        mesh=mesh,
        scratch_types=[
            pltpu.VMEM((_NSR, _K), jnp.int32),
            pltpu.VMEM((_NB, _K), jnp.int32),
            pltpu.VMEM((_NB, _K, d), jnp.float32),
            pltpu.VMEM_SHARED((n, d), jnp.float32),
            pltpu.SemaphoreType.DMA,
            [pltpu.SemaphoreType.DMA] * _NSR,
            [pltpu.SemaphoreType.DMA] * _NB,
            [pltpu.SemaphoreType.DMA] * _NB,
        ],
    )
    def sc_kernel(feat_hbm, edge_hbm, out_hbm,
                  sstage, dstage, rows, acc, semz, ssems, gsems, dsems):
        cid = lax.axis_index("c")
        sid = lax.axis_index("s")
        wid = cid * _NS + sid

        def zfill(i, carry):
            for j in range(d // 16):
                rows[0, i, pl.ds(j * 16, 16)] = jnp.zeros((16,), jnp.float32)
            return carry
        lax.fori_loop(0, _ZR, zfill, 0)

        zsrc = rows.at[0].at[pl.ds(0, _ZR)]

        def zero_chunk(t, carry):
            ch = sid + t * _NS

            @pl.when(ch < nrch)
            def _():
                pltpu.async_copy(zsrc, acc.at[pl.ds(ch * _ZR, _ZR)], semz)
            return carry
        lax.fori_loop(0, nrch_per_tile, zero_chunk, 0)

        def zero_drain(t, carry):
            ch = sid + t * _NS

            @pl.when(ch < nrch)
            def _():
                pltpu.make_async_copy(zsrc, acc.at[pl.ds(ch * _ZR, _ZR)],
                                      semz).wait()
            return carry
        lax.fori_loop(0, nrch_per_tile, zero_drain, 0)

        def sfetch(c, q):
            pltpu.async_copy(edge_hbm.at[wid, c], sstage.at[q], ssems[q])

        def sfetch_wait(c, q):
            pltpu.make_async_copy(edge_hbm.at[wid, c], sstage.at[q],
                                  ssems[q]).wait()

        def gather(c, r, q):
            pltpu.async_copy(feat_hbm.at[sstage.at[q]], rows.at[r], gsems[r])

        def gather_wait(c, r, q):
            pltpu.make_async_copy(feat_hbm.at[sstage.at[q]], rows.at[r],
                                  gsems[r]).wait()

        def dfetch(c, r):
            pltpu.async_copy(edge_hbm.at[_NW + wid, c], dstage.at[r], dsems[r])

        def dfetch_wait(c, r):
            pltpu.make_async_copy(edge_hbm.at[_NW + wid, c], dstage.at[r],
                                  dsems[r]).wait()

        for q in range(_NSR - 1):
            sfetch(q, q)
        for r in range(_NB - 1):
            sfetch_wait(r, r)
            gather(r, r, r)
            dfetch(r, r)
        plsc.subcore_barrier()

        def body(c, carry):
            for q in range(_NSR):
                r = q % _NB

                @pl.when(c % _NSR == q)
                def _():
                    gather_wait(c, r, q)

                    @pl.when(c + _NB - 1 < nchunk)
                    def _():
                        sfetch_wait(c + _NB - 1, (q + _NB - 1) % _NSR)
                        gather(c + _NB - 1, (r + _NB - 1) % _NB,
                               (q + _NB - 1) % _NSR)
                        dfetch(c + _NB - 1, (r + _NB - 1) % _NB)

                    @pl.when(c + _NSR - 1 < nchunk)
                    def _():
                        sfetch(c + _NSR - 1, (q + _NSR - 1) % _NSR)
                    dfetch_wait(c, r)
                    pltpu.sync_copy(rows.at[r], acc.at[dstage.at[r]], add=True)
            return carry
        lax.fori_loop(0, nchunk, body, 0)
        plsc.subcore_barrier()

        def write_chunk(t, carry):
            ch = sid + t * _NS

            @pl.when(ch < nrch)
            def _():
                pltpu.async_copy(acc.at[pl.ds(ch * _ZR, _ZR)],
                                 out_hbm.at[pl.ds(cid * n + ch * _ZR, _ZR)],
                                 semz)
            return carry
        lax.fori_loop(0, nrch_per_tile, write_chunk, 0)

        def write_drain(t, carry):
            ch = sid + t * _NS

            @pl.when(ch < nrch)
            def _():
                pltpu.make_async_copy(
                    acc.at[pl.ds(ch * _ZR, _ZR)],
                    out_hbm.at[pl.ds(cid * n + ch * _ZR, _ZR)], semz).wait()
            return carry
        lax.fori_loop(0, nrch_per_tile, write_drain, 0)

    return sc_kernel(feature, edges3)


def _tc_finish(p3, W, b2, g2, be2):
    _, n, d = p3.shape
    br = 10000
    grid = (n // br,)

    def tc_kernel(p_ref, w_ref, b_ref, g_ref, be_ref, o_ref):
        agg = p_ref[0] + p_ref[1]
        h = lax.dot_general(agg, w_ref[...], (((1,), (1,)), ((), ())),
                            preferred_element_type=jnp.float32)
        h = h + b_ref[...]
        mean = jnp.mean(h, axis=1, keepdims=True)
        cent = h - mean
        var = jnp.mean(cent * cent, axis=1, keepdims=True)
        o_ref[...] = cent * lax.rsqrt(var + _EPS) * g_ref[...] + be_ref[...]

    return pl.pallas_call(
        tc_kernel,
        grid=grid,
        in_specs=[
            pl.BlockSpec((2, br, d), lambda i: (0, i, 0)),
            pl.BlockSpec((d, d), lambda i: (0, 0)),
            pl.BlockSpec((1, d), lambda i: (0, 0)),
            pl.BlockSpec((1, d), lambda i: (0, 0)),
            pl.BlockSpec((1, d), lambda i: (0, 0)),
        ],
        out_specs=pl.BlockSpec((br, d), lambda i: (i, 0)),
        out_shape=jax.ShapeDtypeStruct((n, d), jnp.float32),
    )(p3, W, b2, g2, be2)


def kernel(feature, edge_index, W, b, gamma, beta):
    n, d = feature.shape
    e = edge_index.shape[1]
    ept = e // _NW
    nchunk = ept // _K
    partials = _sc_aggregate(feature,
                             edge_index.reshape(2 * _NW, nchunk, _K))
    return _tc_finish(partials.reshape(_NC, n, d), W,
                      b.reshape(1, d), gamma.reshape(1, d), beta.reshape(1, d))


# speedup vs baseline: 1.0298x; 1.0298x over previous
import functools

import jax
import jax.numpy as jnp
from jax import lax
from jax.experimental import pallas as pl
from jax.experimental.pallas import tpu as pltpu
from jax.experimental.pallas import tpu_sc as plsc

_EPS = 1e-5

_NC = 2
_NS = 16
_NW = _NC * _NS

_K = 80
_ZR = 40
_NB = 4
_NSR = 8


def _sc_aggregate(feature, edges3):
    n, d = feature.shape
    _, nchunk, _ = edges3.shape
    nrch = n // _ZR
    nrch_per_tile = (nrch + _NS - 1) // _NS

    mesh = plsc.VectorSubcoreMesh(core_axis_name="c", subcore_axis_name="s")

    @functools.partial(
        pl.kernel,
        out_type=jax.ShapeDtypeStruct((_NC * n, d), jnp.float32),
        mesh=mesh,
        scratch_types=[
            pltpu.VMEM((_NSR, _K), jnp.int32),
            pltpu.VMEM((_NB, _K), jnp.int32),
            pltpu.VMEM((_NB, _K, d), jnp.float32),
            pltpu.VMEM((_ZR, d), jnp.float32),
            pltpu.VMEM_SHARED((n, d), jnp.float32),
            pltpu.SemaphoreType.DMA,
            [pltpu.SemaphoreType.DMA] * _NSR,
            [pltpu.SemaphoreType.DMA] * _NB,
            [pltpu.SemaphoreType.DMA] * _NB,
        ],
    )
    def sc_kernel(feat_hbm, edge_hbm, out_hbm,
                  sstage, dstage, rows, zbuf, acc, semz, ssems, gsems, dsems):
        cid = lax.axis_index("c")
        sid = lax.axis_index("s")
        wid = cid * _NS + sid

        def sfetch(c, q):
            pltpu.async_copy(edge_hbm.at[wid, c], sstage.at[q], ssems[q])

        def sfetch_wait(c, q):
            pltpu.make_async_copy(edge_hbm.at[wid, c], sstage.at[q],
                                  ssems[q]).wait()

        def gather(c, r, q):
            pltpu.async_copy(feat_hbm.at[sstage.at[q]], rows.at[r], gsems[r])

        def gather_wait(c, r, q):
            pltpu.make_async_copy(feat_hbm.at[sstage.at[q]], rows.at[r],
                                  gsems[r]).wait()

        def dfetch(c, r):
            pltpu.async_copy(edge_hbm.at[_NW + wid, c], dstage.at[r], dsems[r])

        def dfetch_wait(c, r):
            pltpu.make_async_copy(edge_hbm.at[_NW + wid, c], dstage.at[r],
                                  dsems[r]).wait()

        for q in range(_NSR - 1):
            sfetch(q, q)
        for r in range(_NB - 1):
            sfetch_wait(r, r)
            gather(r, r, r)
            dfetch(r, r)

        def zfill(i, carry):
            for j in range(d // 16):
                zbuf[i, pl.ds(j * 16, 16)] = jnp.zeros((16,), jnp.float32)
            return carry
        lax.fori_loop(0, _ZR, zfill, 0)

        zsrc = zbuf

        def zero_chunk(t, carry):
            ch = sid + t * _NS

            @pl.when(ch < nrch)
            def _():
                pltpu.async_copy(zsrc, acc.at[pl.ds(ch * _ZR, _ZR)], semz)
            return carry
        lax.fori_loop(0, nrch_per_tile, zero_chunk, 0)

        def zero_drain(t, carry):
            ch = sid + t * _NS

            @pl.when(ch < nrch)
            def _():
                pltpu.make_async_copy(zsrc, acc.at[pl.ds(ch * _ZR, _ZR)],
                                      semz).wait()
            return carry
        lax.fori_loop(0, nrch_per_tile, zero_drain, 0)

        plsc.subcore_barrier()

        def body(c, carry):
            for q in range(_NSR):
                r = q % _NB

                @pl.when(c % _NSR == q)
                def _():
                    gather_wait(c, r, q)

                    @pl.when(c + _NB - 1 < nchunk)
                    def _():
                        sfetch_wait(c + _NB - 1, (q + _NB - 1) % _NSR)
                        gather(c + _NB - 1, (r + _NB - 1) % _NB,
                               (q + _NB - 1) % _NSR)
                        dfetch(c + _NB - 1, (r + _NB - 1) % _NB)

                    @pl.when(c + _NSR - 1 < nchunk)
                    def _():
                        sfetch(c + _NSR - 1, (q + _NSR - 1) % _NSR)
                    dfetch_wait(c, r)
                    pltpu.sync_copy(rows.at[r], acc.at[dstage.at[r]], add=True)
            return carry
        lax.fori_loop(0, nchunk, body, 0)
        plsc.subcore_barrier()

        def write_chunk(t, carry):
            ch = sid + t * _NS

            @pl.when(ch < nrch)
            def _():
                pltpu.async_copy(acc.at[pl.ds(ch * _ZR, _ZR)],
                                 out_hbm.at[pl.ds(cid * n + ch * _ZR, _ZR)],
                                 semz)
            return carry
        lax.fori_loop(0, nrch_per_tile, write_chunk, 0)

        def write_drain(t, carry):
            ch = sid + t * _NS

            @pl.when(ch < nrch)
            def _():
                pltpu.make_async_copy(
                    acc.at[pl.ds(ch * _ZR, _ZR)],
                    out_hbm.at[pl.ds(cid * n + ch * _ZR, _ZR)], semz).wait()
            return carry
        lax.fori_loop(0, nrch_per_tile, write_drain, 0)

    return sc_kernel(feature, edges3)


def _tc_finish(p3, W, b2, g2, be2):
    _, n, d = p3.shape
    br = 5000
    grid = (n // br,)

    def tc_kernel(p_ref, w_ref, b_ref, g_ref, be_ref, o_ref):
        agg = p_ref[0] + p_ref[1]
        h = lax.dot_general(agg, w_ref[...], (((1,), (1,)), ((), ())),
                            preferred_element_type=jnp.float32)
        h = h + b_ref[...]
        mean = jnp.mean(h, axis=1, keepdims=True)
        cent = h - mean
        var = jnp.mean(cent * cent, axis=1, keepdims=True)
        o_ref[...] = cent * lax.rsqrt(var + _EPS) * g_ref[...] + be_ref[...]

    return pl.pallas_call(
        tc_kernel,
        grid=grid,
        in_specs=[
            pl.BlockSpec((2, br, d), lambda i: (0, i, 0)),
            pl.BlockSpec((d, d), lambda i: (0, 0)),
            pl.BlockSpec((1, d), lambda i: (0, 0)),
            pl.BlockSpec((1, d), lambda i: (0, 0)),
            pl.BlockSpec((1, d), lambda i: (0, 0)),
        ],
        out_specs=pl.BlockSpec((br, d), lambda i: (i, 0)),
        out_shape=jax.ShapeDtypeStruct((n, d), jnp.float32),
    )(p3, W, b2, g2, be2)


def kernel(feature, edge_index, W, b, gamma, beta):
    n, d = feature.shape
    e = edge_index.shape[1]
    ept = e // _NW
    nchunk = ept // _K
    partials = _sc_aggregate(feature,
                             edge_index.reshape(2 * _NW, nchunk, _K))
    return _tc_finish(partials.reshape(_NC, n, d), W,
                      b.reshape(1, d), gamma.reshape(1, d), beta.reshape(1, d))
